# baseline (speedup 1.0000x reference)
import functools

import jax
import jax.numpy as jnp
from jax import lax
from jax.experimental import pallas as pl
from jax.experimental.pallas import tpu as pltpu
from jax.experimental.pallas import tpu_sc as plsc

N = 50000
E = 800000
P = 10
H = 64
HQ = 16
L = 5
G = 128

NS = 16
K = 4
C = K * 128
E_PAD = 819200
NB = E_PAD // 128
NBT = NB // NS
NCH = NBT // K
SCH = 5
NSC = NCH // SCH
NP = 50048
RPT = NP // NS
ZR = 184

BR = 2000
NR = N // BR
EPS = 1e-5

_BCAST_DNUMS = lax.GatherDimensionNumbers(
    offset_dims=(), collapsed_slice_dims=(0,), start_index_map=(0,))


def _lane_bcast(v16, lane):
    idx = jnp.full((16, 1), lane, jnp.int32)
    return lax.gather(v16, idx, _BCAST_DNUMS, (1,),
                      mode=lax.GatherScatterMode.PROMISE_IN_BOUNDS)


def _spmm_body(hs_ref, src3_ref, dst3_ref, w_ref, out_ref,
               rows0, rows1, sstage, dstage, wstage, stage,
               tab, agg, gsem0, gsem1, ssem0, ssem1):
    c = lax.axis_index("c")
    s = lax.axis_index("s")
    rows = (rows0, rows1)
    gsem = (gsem0, gsem1)
    ssem = (ssem0, ssem1)
    z16 = jnp.zeros((16,), jnp.float32)

    def fire_gather(bi, t):
        for j in range(K):
            pltpu.async_copy(tab.at[sstage.at[t * K + j]],
                             rows[bi].at[pl.ds(j * 128, 128), :], gsem[bi])

    def drain_gather(bi, t):
        for j in range(K):
            pltpu.make_async_copy(
                tab.at[sstage.at[t * K + j]],
                rows[bi].at[pl.ds(j * 128, 128), :], gsem[bi]).wait()

    def fire_scatter(bi, t):
        for j in range(K):
            pltpu.async_copy(rows[bi].at[pl.ds(j * 128, 128), :],
                             agg.at[dstage.at[t * K + j]], ssem[bi], add=True)

    def drain_scatter(bi, t):
        for j in range(K):
            pltpu.make_async_copy(
                rows[bi].at[pl.ds(j * 128, 128), :],
                agg.at[dstage.at[t * K + j]], ssem[bi]).wait()

    def multiply(bi, t):
        rw = rows[bi]
        woff = t * C

        @plsc.parallel_loop(0, C // 16, 1, unroll=2)
        def _(g):
            w16 = wstage[pl.ds(woff + g * 16, 16)]
            for e in range(16):
                wv = _lane_bcast(w16, e)
                r = g * 16 + e
                rw[r, pl.ds(0, 16)] = rw[r, pl.ds(0, 16)] * wv

    for p in range(2):
        pltpu.sync_copy(hs_ref.at[2 * c + p, pl.ds(s * RPT, RPT), :],
                        tab.at[pl.ds(s * RPT, RPT), :])

        def zloop(r, _):
            stage[r, pl.ds(0, 16)] = z16
            return 0

        lax.fori_loop(0, ZR, zloop, 0)
        for q in range(RPT // ZR):
            pltpu.sync_copy(stage, agg.at[pl.ds(s * RPT + q * ZR, ZR), :])
        plsc.subcore_barrier()

        def superchunk(sc, _):
            base_blk = s * NBT + sc * (SCH * K)
            pltpu.sync_copy(src3_ref.at[pl.ds(base_blk, SCH * K), :], sstage)
            pltpu.sync_copy(dst3_ref.at[pl.ds(base_blk, SCH * K), :], dstage)
            pltpu.sync_copy(w_ref.at[pl.ds(base_blk * 128, SCH * C)], wstage)

            def pair(q, _):
                for b in range(2):
                    t2 = q * 2 + b

                    @pl.when(jnp.logical_and(t2 >= 2, t2 < SCH))
                    def _():
                        drain_scatter(b, t2 - 2)

                    @pl.when(t2 < SCH)
                    def _():
                        fire_gather(b, t2)

                    @pl.when(jnp.logical_and(t2 >= 1, t2 <= SCH))
                    def _():
                        drain_gather(1 - b, t2 - 1)
                        multiply(1 - b, t2 - 1)
                        fire_scatter(1 - b, t2 - 1)
                return 0

            lax.fori_loop(0, SCH // 2 + 1, pair, 0)
            drain_scatter(0, SCH - 2)
            drain_scatter(1, SCH - 1)
            return 0

        lax.fori_loop(0, NSC, superchunk, 0)
        plsc.subcore_barrier()

        for q in range(RPT // ZR):
            base = s * RPT + q * ZR
            pltpu.sync_copy(agg.at[pl.ds(base, ZR), :], stage)
            pltpu.sync_copy(stage, out_ref.at[2 * c + p, pl.ds(base, ZR), :])
        plsc.subcore_barrier()


@functools.cache
def _spmm_kernel():
    return pl.kernel(
        _spmm_body,
        out_type=jax.ShapeDtypeStruct((4, NP, HQ), jnp.float32),
        mesh=plsc.VectorSubcoreMesh(core_axis_name="c", subcore_axis_name="s"),
        scratch_types=[
            pltpu.VMEM((C, HQ), jnp.float32),
            pltpu.VMEM((C, HQ), jnp.float32),
            pltpu.VMEM((SCH * K, 128), jnp.int32),
            pltpu.VMEM((SCH * K, 128), jnp.int32),
            pltpu.VMEM((SCH * C,), jnp.float32),
            pltpu.VMEM((ZR, HQ), jnp.float32),
            pltpu.VMEM_SHARED((NP, HQ), jnp.float32),
            pltpu.VMEM_SHARED((NP, HQ), jnp.float32),
            pltpu.SemaphoreType.DMA,
            pltpu.SemaphoreType.DMA,
            pltpu.SemaphoreType.DMA,
            pltpu.SemaphoreType.DMA,
        ],
        compiler_params=pltpu.CompilerParams(use_tc_tiling_on_sc=False),
    )


def _spmm(hs, src3, dst3, wp):
    return _spmm_kernel()(hs, src3, dst3, wp)


def _lin_body(x_ref, w_ref, b_ref, o_ref):
    h = jnp.dot(x_ref[...], w_ref[...],
                preferred_element_type=jnp.float32) + b_ref[...]
    for k in range(4):
        o_ref[k] = h[:, k * HQ:(k + 1) * HQ]


def _lin(x, w, b):
    return pl.pallas_call(
        _lin_body,
        grid=(NR,),
        in_specs=[
            pl.BlockSpec((BR, 2 * P), lambda i: (i, 0)),
            pl.BlockSpec((2 * P, H), lambda i: (0, 0)),
            pl.BlockSpec((1, H), lambda i: (0, 0)),
        ],
        out_specs=pl.BlockSpec((4, BR, HQ), lambda i: (0, i, 0)),
        out_shape=jax.ShapeDtypeStruct((4, NP, HQ), jnp.float32),
    )(x, w, b)


def _t1_body(agg_ref, w_ref, b_ref, t_ref, st_ref, acc):
    i = pl.program_id(0)
    a = jnp.concatenate([agg_ref[k] for k in range(4)], axis=1)
    t = jnp.dot(a, w_ref[...], preferred_element_type=jnp.float32) + b_ref[...]
    t_ref[...] = t

    @pl.when(i == 0)
    def _():
        acc[...] = jnp.zeros_like(acc)

    acc[0:1, :] += jnp.sum(t, axis=0, keepdims=True)
    acc[1:2, :] += jnp.sum(t * t, axis=0, keepdims=True)

    @pl.when(i == NR - 1)
    def _():
        st_ref[...] = acc[...]


def _t1(agg, w, b):
    return pl.pallas_call(
        _t1_body,
        grid=(NR,),
        in_specs=[
            pl.BlockSpec((4, BR, HQ), lambda i: (0, i, 0)),
            pl.BlockSpec((H, 2 * H), lambda i: (0, 0)),
            pl.BlockSpec((1, 2 * H), lambda i: (0, 0)),
        ],
        out_specs=[
            pl.BlockSpec((BR, 2 * H), lambda i: (i, 0)),
            pl.BlockSpec((2, 2 * H), lambda i: (0, 0)),
        ],
        out_shape=[
            jax.ShapeDtypeStruct((N, 2 * H), jnp.float32),
            jax.ShapeDtypeStruct((2, 2 * H), jnp.float32),
        ],
        scratch_shapes=[pltpu.VMEM((2, 2 * H), jnp.float32)],
    )(agg, w, b)


def _t2_body(t_ref, st_ref, g_ref, be_ref, w_ref, b_ref, u_ref, su_ref, acc):
    i = pl.program_id(0)
    st = st_ref[...]
    mean = st[0:1, :] * (1.0 / N)
    var = st[1:2, :] * (1.0 / N) - mean * mean
    scale = g_ref[...] * lax.rsqrt(var + EPS)
    shift = be_ref[...] - mean * scale
    tn = jnp.maximum(t_ref[...] * scale + shift, 0.0)
    u = jnp.dot(tn, w_ref[...], preferred_element_type=jnp.float32) + b_ref[...]
    for k in range(4):
        u_ref[k] = u[:, k * HQ:(k + 1) * HQ]

    @pl.when(i == 0)
    def _():
        acc[...] = jnp.zeros_like(acc)

    acc[0:1, :] += jnp.sum(u, axis=0, keepdims=True)
    acc[1:2, :] += jnp.sum(u * u, axis=0, keepdims=True)

    @pl.when(i == NR - 1)
    def _():
        su_ref[...] = acc[...]


def _t2(t, st, g, be, w, b):
    return pl.pallas_call(
        _t2_body,
        grid=(NR,),
        in_specs=[
            pl.BlockSpec((BR, 2 * H), lambda i: (i, 0)),
            pl.BlockSpec((2, 2 * H), lambda i: (0, 0)),
            pl.BlockSpec((1, 2 * H), lambda i: (0, 0)),
            pl.BlockSpec((1, 2 * H), lambda i: (0, 0)),
            pl.BlockSpec((2 * H, H), lambda i: (0, 0)),
            pl.BlockSpec((1, H), lambda i: (0, 0)),
        ],
        out_specs=[
            pl.BlockSpec((4, BR, HQ), lambda i: (0, i, 0)),
            pl.BlockSpec((2, H), lambda i: (0, 0)),
        ],
        out_shape=[
            jax.ShapeDtypeStruct((4, NP, HQ), jnp.float32),
            jax.ShapeDtypeStruct((2, H), jnp.float32),
        ],
        scratch_shapes=[pltpu.VMEM((2, H), jnp.float32)],
    )(t, st, g, be, w, b)


def _t3_body(u_ref, su_ref, g_ref, be_ref, o_ref):
    su = su_ref[...]
    mean = su[0:1, :] * (1.0 / N)
    var = su[1:2, :] * (1.0 / N) - mean * mean
    scale = g_ref[...] * lax.rsqrt(var + EPS)
    shift = be_ref[...] - mean * scale
    for k in range(4):
        sl = scale[:, k * HQ:(k + 1) * HQ]
        sh = shift[:, k * HQ:(k + 1) * HQ]
        o_ref[k] = jnp.maximum(u_ref[k] * sl + sh, 0.0)


def _t3(u, su, g, be):
    return pl.pallas_call(
        _t3_body,
        grid=(NR,),
        in_specs=[
            pl.BlockSpec((4, BR, HQ), lambda i: (0, i, 0)),
            pl.BlockSpec((2, H), lambda i: (0, 0)),
            pl.BlockSpec((1, H), lambda i: (0, 0)),
            pl.BlockSpec((1, H), lambda i: (0, 0)),
        ],
        out_specs=pl.BlockSpec((4, BR, HQ), lambda i: (0, i, 0)),
        out_shape=jax.ShapeDtypeStruct((4, NP, HQ), jnp.float32),
    )(u, su, g, be)


def _t3f_body(u_ref, su_ref, g_ref, be_ref, o_ref):
    su = su_ref[...]
    mean = su[0:1, :] * (1.0 / N)
    var = su[1:2, :] * (1.0 / N) - mean * mean
    scale = g_ref[...] * lax.rsqrt(var + EPS)
    shift = be_ref[...] - mean * scale
    un = jnp.concatenate([u_ref[k] for k in range(4)], axis=1)
    o_ref[...] = un * scale + shift


def _t3f(u, su, g, be):
    return pl.pallas_call(
        _t3f_body,
        grid=(NR,),
        in_specs=[
            pl.BlockSpec((4, BR, HQ), lambda i: (0, i, 0)),
            pl.BlockSpec((2, H), lambda i: (0, 0)),
            pl.BlockSpec((1, H), lambda i: (0, 0)),
            pl.BlockSpec((1, H), lambda i: (0, 0)),
        ],
        out_specs=pl.BlockSpec((BR, H), lambda i: (i, 0)),
        out_shape=jax.ShapeDtypeStruct((N, H), jnp.float32),
    )(u, su, g, be)


def _pool_body(h_ref, b_ref, o_ref):
    i = pl.program_id(0)
    bb = b_ref[0]
    onehot_t = (lax.broadcasted_iota(jnp.int32, (G, BR), 0) == bb).astype(
        jnp.float32)

    @pl.when(i == 0)
    def _():
        o_ref[...] = jnp.zeros_like(o_ref)

    o_ref[...] += jnp.dot(onehot_t, h_ref[...],
                          preferred_element_type=jnp.float32)


def _pool(h, batch3):
    return pl.pallas_call(
        _pool_body,
        grid=(NR,),
        in_specs=[
            pl.BlockSpec((BR, H), lambda i: (i, 0)),
            pl.BlockSpec((1, 1, BR), lambda i: (i, 0, 0)),
        ],
        out_specs=pl.BlockSpec((G, H), lambda i: (0, 0)),
        out_shape=jax.ShapeDtypeStruct((G, H), jnp.float32),
    )(h, batch3)


def kernel(x, edge_attr, lin_w, lin_b, W1, b1, g1, be1, W2, b2, gO, beO,
           batch, edge_index):
    src = edge_index[0]
    dst = edge_index[1]
    pad = E_PAD - E
    srcp = jnp.pad(src, (0, pad))
    dstp = jnp.pad(dst, (0, pad))
    wp = jnp.pad(edge_attr, (0, pad))
    src3 = srcp.reshape(NB, 128)
    dst3 = dstp.reshape(NB, 128)
    batch3 = batch.reshape(NR, 1, BR)

    hs = _lin(x, lin_w, lin_b.reshape(1, H))
    h = None
    for i in range(L):
        agg = _spmm(hs, src3, dst3, wp)
        t, st = _t1(agg, W1[i], b1[i].reshape(1, 2 * H))
        u, su = _t2(t, st, g1[i].reshape(1, 2 * H), be1[i].reshape(1, 2 * H),
                    W2[i], b2[i].reshape(1, H))
        if i < L - 1:
            hs = _t3(u, su, gO[i].reshape(1, H), beO[i].reshape(1, H))
        else:
            h = _t3f(u, su, gO[i].reshape(1, H), beO[i].reshape(1, H))
    xpool = _pool(h, batch3)
    return h, xpool


# speedup vs baseline: 1.1022x; 1.1022x over previous
import functools

import jax
import jax.numpy as jnp
from jax import lax
from jax.experimental import pallas as pl
from jax.experimental.pallas import tpu as pltpu
from jax.experimental.pallas import tpu_sc as plsc

N = 50000
E = 800000
P = 10
H = 64
HQ = 16
L = 5
G = 128

NS = 16
K = 2
C = K * 128
E_PAD = 819200
NB = E_PAD // 128
NBT = NB // NS
NCH = NBT // K
SCH = 20
NSC = NCH // SCH
NP = 50048
RPT = NP // NS
ZR = 184

BR = 2000
NR = N // BR
EPS = 1e-5

_BCAST_DNUMS = lax.GatherDimensionNumbers(
    offset_dims=(), collapsed_slice_dims=(0,), start_index_map=(0,))


def _lane_bcast(v16, lane):
    idx = jnp.full((16, 1), lane, jnp.int32)
    return lax.gather(v16, idx, _BCAST_DNUMS, (1,),
                      mode=lax.GatherScatterMode.PROMISE_IN_BOUNDS)


def _spmm_body(hs_ref, src3_ref, dst3_ref, w_ref, out_ref,
               rows0, rows1, sstage, dstage, wstage, stage,
               tab, agg, gsem0, gsem1, ssem0, ssem1):
    c = lax.axis_index("c")
    s = lax.axis_index("s")
    rows = (rows0, rows1)
    gsem = (gsem0, gsem1)
    ssem = (ssem0, ssem1)
    z16 = jnp.zeros((16,), jnp.float32)

    def fire_gather(bi, t):
        for j in range(K):
            pltpu.async_copy(tab.at[sstage.at[t * K + j]],
                             rows[bi].at[pl.ds(j * 128, 128), :], gsem[bi])

    def drain_gather(bi, t):
        for j in range(K):
            pltpu.make_async_copy(
                tab.at[sstage.at[t * K + j]],
                rows[bi].at[pl.ds(j * 128, 128), :], gsem[bi]).wait()

    def fire_scatter(bi, t):
        for j in range(K):
            pltpu.async_copy(rows[bi].at[pl.ds(j * 128, 128), :],
                             agg.at[dstage.at[t * K + j]], ssem[bi], add=True)

    def drain_scatter(bi, t):
        for j in range(K):
            pltpu.make_async_copy(
                rows[bi].at[pl.ds(j * 128, 128), :],
                agg.at[dstage.at[t * K + j]], ssem[bi]).wait()

    def multiply(bi, t):
        rw = rows[bi]
        woff = t * C

        @plsc.parallel_loop(0, C // 16, 1, unroll=2)
        def _(g):
            w16 = wstage[pl.ds(woff + g * 16, 16)]
            for e in range(16):
                wv = _lane_bcast(w16, e)
                r = g * 16 + e
                rw[r, pl.ds(0, 16)] = rw[r, pl.ds(0, 16)] * wv

    for p in range(2):
        pltpu.sync_copy(hs_ref.at[2 * c + p, pl.ds(s * RPT, RPT), :],
                        tab.at[pl.ds(s * RPT, RPT), :])

        def zloop(r, _):
            stage[r, pl.ds(0, 16)] = z16
            return 0

        lax.fori_loop(0, ZR, zloop, 0)
        for q in range(RPT // ZR):
            pltpu.sync_copy(stage, agg.at[pl.ds(s * RPT + q * ZR, ZR), :])
        plsc.subcore_barrier()

        def superchunk(sc, _):
            base_blk = s * NBT + sc * (SCH * K)
            pltpu.sync_copy(src3_ref.at[pl.ds(base_blk, SCH * K), :], sstage)
            pltpu.sync_copy(dst3_ref.at[pl.ds(base_blk, SCH * K), :], dstage)
            pltpu.sync_copy(w_ref.at[pl.ds(base_blk * 128, SCH * C)], wstage)

            def pair(q, _):
                for b in range(2):
                    t2 = q * 2 + b

                    @pl.when(jnp.logical_and(t2 >= 2, t2 < SCH))
                    def _():
                        drain_scatter(b, t2 - 2)

                    @pl.when(t2 < SCH)
                    def _():
                        fire_gather(b, t2)

                    @pl.when(jnp.logical_and(t2 >= 1, t2 <= SCH))
                    def _():
                        drain_gather(1 - b, t2 - 1)
                        multiply(1 - b, t2 - 1)
                        fire_scatter(1 - b, t2 - 1)
                return 0

            lax.fori_loop(0, SCH // 2 + 1, pair, 0)
            drain_scatter(0, SCH - 2)
            drain_scatter(1, SCH - 1)
            return 0

        lax.fori_loop(0, NSC, superchunk, 0)
        plsc.subcore_barrier()

        for q in range(RPT // ZR):
            base = s * RPT + q * ZR
            pltpu.sync_copy(agg.at[pl.ds(base, ZR), :], stage)
            pltpu.sync_copy(stage, out_ref.at[2 * c + p, pl.ds(base, ZR), :])
        plsc.subcore_barrier()


@functools.cache
def _spmm_kernel():
    return pl.kernel(
        _spmm_body,
        out_type=jax.ShapeDtypeStruct((4, NP, HQ), jnp.float32),
        mesh=plsc.VectorSubcoreMesh(core_axis_name="c", subcore_axis_name="s"),
        scratch_types=[
            pltpu.VMEM((C, HQ), jnp.float32),
            pltpu.VMEM((C, HQ), jnp.float32),
            pltpu.VMEM((SCH * K, 128), jnp.int32),
            pltpu.VMEM((SCH * K, 128), jnp.int32),
            pltpu.VMEM((SCH * C,), jnp.float32),
            pltpu.VMEM((ZR, HQ), jnp.float32),
            pltpu.VMEM_SHARED((NP, HQ), jnp.float32),
            pltpu.VMEM_SHARED((NP, HQ), jnp.float32),
            pltpu.SemaphoreType.DMA,
            pltpu.SemaphoreType.DMA,
            pltpu.SemaphoreType.DMA,
            pltpu.SemaphoreType.DMA,
        ],
        compiler_params=pltpu.CompilerParams(use_tc_tiling_on_sc=False),
    )


def _spmm(hs, src3, dst3, wp):
    return _spmm_kernel()(hs, src3, dst3, wp)


def _lin_body(x_ref, w_ref, b_ref, o_ref):
    h = jnp.dot(x_ref[...], w_ref[...],
                preferred_element_type=jnp.float32) + b_ref[...]
    for k in range(4):
        o_ref[k] = h[:, k * HQ:(k + 1) * HQ]


def _lin(x, w, b):
    return pl.pallas_call(
        _lin_body,
        grid=(NR,),
        in_specs=[
            pl.BlockSpec((BR, 2 * P), lambda i: (i, 0)),
            pl.BlockSpec((2 * P, H), lambda i: (0, 0)),
            pl.BlockSpec((1, H), lambda i: (0, 0)),
        ],
        out_specs=pl.BlockSpec((4, BR, HQ), lambda i: (0, i, 0)),
        out_shape=jax.ShapeDtypeStruct((4, NP, HQ), jnp.float32),
    )(x, w, b)


def _t1_body(agg_ref, w_ref, b_ref, t_ref, st_ref, acc):
    i = pl.program_id(0)
    a = jnp.concatenate([agg_ref[k] for k in range(4)], axis=1)
    t = jnp.dot(a, w_ref[...], preferred_element_type=jnp.float32) + b_ref[...]
    t_ref[...] = t

    @pl.when(i == 0)
    def _():
        acc[...] = jnp.zeros_like(acc)

    acc[0:1, :] += jnp.sum(t, axis=0, keepdims=True)
    acc[1:2, :] += jnp.sum(t * t, axis=0, keepdims=True)

    @pl.when(i == NR - 1)
    def _():
        st_ref[...] = acc[...]


def _t1(agg, w, b):
    return pl.pallas_call(
        _t1_body,
        grid=(NR,),
        in_specs=[
            pl.BlockSpec((4, BR, HQ), lambda i: (0, i, 0)),
            pl.BlockSpec((H, 2 * H), lambda i: (0, 0)),
            pl.BlockSpec((1, 2 * H), lambda i: (0, 0)),
        ],
        out_specs=[
            pl.BlockSpec((BR, 2 * H), lambda i: (i, 0)),
            pl.BlockSpec((2, 2 * H), lambda i: (0, 0)),
        ],
        out_shape=[
            jax.ShapeDtypeStruct((N, 2 * H), jnp.float32),
            jax.ShapeDtypeStruct((2, 2 * H), jnp.float32),
        ],
        scratch_shapes=[pltpu.VMEM((2, 2 * H), jnp.float32)],
    )(agg, w, b)


def _t2_body(t_ref, st_ref, g_ref, be_ref, w_ref, b_ref, u_ref, su_ref, acc):
    i = pl.program_id(0)
    st = st_ref[...]
    mean = st[0:1, :] * (1.0 / N)
    var = st[1:2, :] * (1.0 / N) - mean * mean
    scale = g_ref[...] * lax.rsqrt(var + EPS)
    shift = be_ref[...] - mean * scale
    tn = jnp.maximum(t_ref[...] * scale + shift, 0.0)
    u = jnp.dot(tn, w_ref[...], preferred_element_type=jnp.float32) + b_ref[...]
    for k in range(4):
        u_ref[k] = u[:, k * HQ:(k + 1) * HQ]

    @pl.when(i == 0)
    def _():
        acc[...] = jnp.zeros_like(acc)

    acc[0:1, :] += jnp.sum(u, axis=0, keepdims=True)
    acc[1:2, :] += jnp.sum(u * u, axis=0, keepdims=True)

    @pl.when(i == NR - 1)
    def _():
        su_ref[...] = acc[...]


def _t2(t, st, g, be, w, b):
    return pl.pallas_call(
        _t2_body,
        grid=(NR,),
        in_specs=[
            pl.BlockSpec((BR, 2 * H), lambda i: (i, 0)),
            pl.BlockSpec((2, 2 * H), lambda i: (0, 0)),
            pl.BlockSpec((1, 2 * H), lambda i: (0, 0)),
            pl.BlockSpec((1, 2 * H), lambda i: (0, 0)),
            pl.BlockSpec((2 * H, H), lambda i: (0, 0)),
            pl.BlockSpec((1, H), lambda i: (0, 0)),
        ],
        out_specs=[
            pl.BlockSpec((4, BR, HQ), lambda i: (0, i, 0)),
            pl.BlockSpec((2, H), lambda i: (0, 0)),
        ],
        out_shape=[
            jax.ShapeDtypeStruct((4, NP, HQ), jnp.float32),
            jax.ShapeDtypeStruct((2, H), jnp.float32),
        ],
        scratch_shapes=[pltpu.VMEM((2, H), jnp.float32)],
    )(t, st, g, be, w, b)


def _t3_body(u_ref, su_ref, g_ref, be_ref, o_ref):
    su = su_ref[...]
    mean = su[0:1, :] * (1.0 / N)
    var = su[1:2, :] * (1.0 / N) - mean * mean
    scale = g_ref[...] * lax.rsqrt(var + EPS)
    shift = be_ref[...] - mean * scale
    for k in range(4):
        sl = scale[:, k * HQ:(k + 1) * HQ]
        sh = shift[:, k * HQ:(k + 1) * HQ]
        o_ref[k] = jnp.maximum(u_ref[k] * sl + sh, 0.0)


def _t3(u, su, g, be):
    return pl.pallas_call(
        _t3_body,
        grid=(NR,),
        in_specs=[
            pl.BlockSpec((4, BR, HQ), lambda i: (0, i, 0)),
            pl.BlockSpec((2, H), lambda i: (0, 0)),
            pl.BlockSpec((1, H), lambda i: (0, 0)),
            pl.BlockSpec((1, H), lambda i: (0, 0)),
        ],
        out_specs=pl.BlockSpec((4, BR, HQ), lambda i: (0, i, 0)),
        out_shape=jax.ShapeDtypeStruct((4, NP, HQ), jnp.float32),
    )(u, su, g, be)


def _t3f_body(u_ref, su_ref, g_ref, be_ref, o_ref):
    su = su_ref[...]
    mean = su[0:1, :] * (1.0 / N)
    var = su[1:2, :] * (1.0 / N) - mean * mean
    scale = g_ref[...] * lax.rsqrt(var + EPS)
    shift = be_ref[...] - mean * scale
    un = jnp.concatenate([u_ref[k] for k in range(4)], axis=1)
    o_ref[...] = un * scale + shift


def _t3f(u, su, g, be):
    return pl.pallas_call(
        _t3f_body,
        grid=(NR,),
        in_specs=[
            pl.BlockSpec((4, BR, HQ), lambda i: (0, i, 0)),
            pl.BlockSpec((2, H), lambda i: (0, 0)),
            pl.BlockSpec((1, H), lambda i: (0, 0)),
            pl.BlockSpec((1, H), lambda i: (0, 0)),
        ],
        out_specs=pl.BlockSpec((BR, H), lambda i: (i, 0)),
        out_shape=jax.ShapeDtypeStruct((N, H), jnp.float32),
    )(u, su, g, be)


def _pool_body(h_ref, b_ref, o_ref):
    i = pl.program_id(0)
    bb = b_ref[0]
    onehot_t = (lax.broadcasted_iota(jnp.int32, (G, BR), 0) == bb).astype(
        jnp.float32)

    @pl.when(i == 0)
    def _():
        o_ref[...] = jnp.zeros_like(o_ref)

    o_ref[...] += jnp.dot(onehot_t, h_ref[...],
                          preferred_element_type=jnp.float32)


def _pool(h, batch3):
    return pl.pallas_call(
        _pool_body,
        grid=(NR,),
        in_specs=[
            pl.BlockSpec((BR, H), lambda i: (i, 0)),
            pl.BlockSpec((1, 1, BR), lambda i: (i, 0, 0)),
        ],
        out_specs=pl.BlockSpec((G, H), lambda i: (0, 0)),
        out_shape=jax.ShapeDtypeStruct((G, H), jnp.float32),
    )(h, batch3)


def kernel(x, edge_attr, lin_w, lin_b, W1, b1, g1, be1, W2, b2, gO, beO,
           batch, edge_index):
    src = edge_index[0]
    dst = edge_index[1]
    pad = E_PAD - E
    srcp = jnp.pad(src, (0, pad))
    dstp = jnp.pad(dst, (0, pad))
    wp = jnp.pad(edge_attr, (0, pad))
    src3 = srcp.reshape(NB, 128)
    dst3 = dstp.reshape(NB, 128)
    batch3 = batch.reshape(NR, 1, BR)

    hs = _lin(x, lin_w, lin_b.reshape(1, H))
    h = None
    for i in range(L):
        agg = _spmm(hs, src3, dst3, wp)
        t, st = _t1(agg, W1[i], b1[i].reshape(1, 2 * H))
        u, su = _t2(t, st, g1[i].reshape(1, 2 * H), be1[i].reshape(1, 2 * H),
                    W2[i], b2[i].reshape(1, H))
        if i < L - 1:
            hs = _t3(u, su, gO[i].reshape(1, H), beO[i].reshape(1, H))
        else:
            h = _t3f(u, su, gO[i].reshape(1, H), beO[i].reshape(1, H))
    xpool = _pool(h, batch3)
    return h, xpool


# speedup vs baseline: 1.1911x; 1.0807x over previous
import functools

import jax
import jax.numpy as jnp
from jax import lax
from jax.experimental import pallas as pl
from jax.experimental.pallas import tpu as pltpu
from jax.experimental.pallas import tpu_sc as plsc

N = 50000
E = 800000
P = 10
H = 64
HQ = 16
L = 5
G = 128

NS = 16
K = 2
C = K * 128
E_PAD = 819200
NB = E_PAD // 128
NBT = NB // NS
NCH = NBT // K
SCH = 20
NSC = NCH // SCH
NP = 50048
RPT = NP // NS
ZR = 184

BR = 2000
NR = N // BR
EPS = 1e-5

_BCAST_DNUMS = lax.GatherDimensionNumbers(
    offset_dims=(), collapsed_slice_dims=(0,), start_index_map=(0,))


def _lane_bcast(v16, lane):
    idx = jnp.full((16, 1), lane, jnp.int32)
    return lax.gather(v16, idx, _BCAST_DNUMS, (1,),
                      mode=lax.GatherScatterMode.PROMISE_IN_BOUNDS)


def _spmm_body(hs_ref, src3_ref, dst3_ref, w_ref, out_ref,
               rows0, rows1, sstage, dstage, wstage, stage,
               tab, agg, gsem0, gsem1, ssem0, ssem1):
    c = lax.axis_index("c")
    s = lax.axis_index("s")
    rows = (rows0, rows1)
    gsem = (gsem0, gsem1)
    ssem = (ssem0, ssem1)
    z16 = jnp.zeros((16,), jnp.float32)

    def fire_gather(bi, t):
        for j in range(K):
            pltpu.async_copy(tab.at[sstage.at[t * K + j]],
                             rows[bi].at[pl.ds(j * 128, 128), :], gsem[bi])

    def drain_gather(bi, t):
        for j in range(K):
            pltpu.make_async_copy(
                tab.at[sstage.at[t * K + j]],
                rows[bi].at[pl.ds(j * 128, 128), :], gsem[bi]).wait()

    def fire_scatter(bi, t):
        for j in range(K):
            pltpu.async_copy(rows[bi].at[pl.ds(j * 128, 128), :],
                             agg.at[dstage.at[t * K + j]], ssem[bi], add=True)

    def drain_scatter(bi, t):
        for j in range(K):
            pltpu.make_async_copy(
                rows[bi].at[pl.ds(j * 128, 128), :],
                agg.at[dstage.at[t * K + j]], ssem[bi]).wait()

    def multiply(bi, t):
        rw = rows[bi]
        woff = t * C

        @plsc.parallel_loop(0, C // 16, 1, unroll=2)
        def _(g):
            w16 = wstage[pl.ds(woff + g * 16, 16)]
            for e in range(16):
                wv = _lane_bcast(w16, e)
                r = g * 16 + e
                rw[r, pl.ds(0, 16)] = rw[r, pl.ds(0, 16)] * wv

    for p in range(2):
        pltpu.sync_copy(hs_ref.at[2 * c + p, pl.ds(s * RPT, RPT), :],
                        tab.at[pl.ds(s * RPT, RPT), :])

        def zloop(r, _):
            stage[r, pl.ds(0, 16)] = z16
            return 0

        lax.fori_loop(0, ZR, zloop, 0)
        for q in range(RPT // ZR):
            pltpu.sync_copy(stage, agg.at[pl.ds(s * RPT + q * ZR, ZR), :])
        plsc.subcore_barrier()

        def superchunk(sc, _):
            base_blk = s * NBT + sc * (SCH * K)
            pltpu.sync_copy(src3_ref.at[pl.ds(base_blk, SCH * K), :], sstage)
            pltpu.sync_copy(dst3_ref.at[pl.ds(base_blk, SCH * K), :], dstage)
            pltpu.sync_copy(w_ref.at[pl.ds(base_blk * 128, SCH * C)], wstage)

            def pair(q, _):
                for b in range(2):
                    t2 = q * 2 + b

                    @pl.when(jnp.logical_and(t2 >= 2, t2 < SCH))
                    def _():
                        drain_scatter(b, t2 - 2)

                    @pl.when(t2 < SCH)
                    def _():
                        fire_gather(b, t2)

                    @pl.when(jnp.logical_and(t2 >= 1, t2 <= SCH))
                    def _():
                        drain_gather(1 - b, t2 - 1)
                        multiply(1 - b, t2 - 1)
                        fire_scatter(1 - b, t2 - 1)
                return 0

            lax.fori_loop(0, SCH // 2 + 1, pair, 0)
            drain_scatter(0, SCH - 2)
            drain_scatter(1, SCH - 1)
            return 0

        lax.fori_loop(0, NSC, superchunk, 0)
        plsc.subcore_barrier()

        for q in range(RPT // ZR):
            base = s * RPT + q * ZR
            pltpu.sync_copy(agg.at[pl.ds(base, ZR), :], stage)
            pltpu.sync_copy(stage, out_ref.at[2 * c + p, pl.ds(base, ZR), :])
        plsc.subcore_barrier()


@functools.cache
def _spmm_kernel():
    return pl.kernel(
        _spmm_body,
        out_type=jax.ShapeDtypeStruct((4, NP, HQ), jnp.float32),
        mesh=plsc.VectorSubcoreMesh(core_axis_name="c", subcore_axis_name="s"),
        scratch_types=[
            pltpu.VMEM((C, HQ), jnp.float32),
            pltpu.VMEM((C, HQ), jnp.float32),
            pltpu.VMEM((SCH * K, 128), jnp.int32),
            pltpu.VMEM((SCH * K, 128), jnp.int32),
            pltpu.VMEM((SCH * C,), jnp.float32),
            pltpu.VMEM((ZR, HQ), jnp.float32),
            pltpu.VMEM_SHARED((NP, HQ), jnp.float32),
            pltpu.VMEM_SHARED((NP, HQ), jnp.float32),
            pltpu.SemaphoreType.DMA,
            pltpu.SemaphoreType.DMA,
            pltpu.SemaphoreType.DMA,
            pltpu.SemaphoreType.DMA,
        ],
        compiler_params=pltpu.CompilerParams(use_tc_tiling_on_sc=False),
    )


def _spmm(hs, src3, dst3, wp):
    return _spmm_kernel()(hs, src3, dst3, wp)


def _lin_body(x_ref, w_ref, b_ref, o_ref):
    h = jnp.dot(x_ref[...], w_ref[...],
                preferred_element_type=jnp.float32) + b_ref[...]
    for k in range(4):
        o_ref[k] = h[:, k * HQ:(k + 1) * HQ]


def _lin(x, w, b):
    return pl.pallas_call(
        _lin_body,
        grid=(NR,),
        in_specs=[
            pl.BlockSpec((BR, 2 * P), lambda i: (i, 0)),
            pl.BlockSpec((2 * P, H), lambda i: (0, 0)),
            pl.BlockSpec((1, H), lambda i: (0, 0)),
        ],
        out_specs=pl.BlockSpec((4, BR, HQ), lambda i: (0, i, 0)),
        out_shape=jax.ShapeDtypeStruct((4, NP, HQ), jnp.float32),
    )(x, w, b)


def _mlp_body_maker(final):
    def body(*refs):
        if final:
            (agg_ref, w1_ref, b1_ref, g1_ref, be1_ref, w2_ref, b2_ref,
             go_ref, beo_ref, bat_ref, h_ref, xp_ref, us, st, su) = refs
        else:
            (agg_ref, w1_ref, b1_ref, g1_ref, be1_ref, w2_ref, b2_ref,
             go_ref, beo_ref, o_ref, us, st, su) = refs
        p = pl.program_id(0)
        i = pl.program_id(1)

        def compute_t():
            a = jnp.concatenate([agg_ref[k] for k in range(4)], axis=1)
            return jnp.dot(a, w1_ref[...],
                           preferred_element_type=jnp.float32) + b1_ref[...]

        @pl.when(p == 0)
        def _():
            t = compute_t()

            @pl.when(i == 0)
            def _():
                st[...] = jnp.zeros_like(st)

            st[0:1, :] += jnp.sum(t, axis=0, keepdims=True)
            st[1:2, :] += jnp.sum(t * t, axis=0, keepdims=True)

        @pl.when(p == 1)
        def _():
            t = compute_t()
            sv = st[...]
            mean = sv[0:1, :] * (1.0 / N)
            var = sv[1:2, :] * (1.0 / N) - mean * mean
            scale = g1_ref[...] * lax.rsqrt(var + EPS)
            shift = be1_ref[...] - mean * scale
            tn = jnp.maximum(t * scale + shift, 0.0)
            u = jnp.dot(tn, w2_ref[...],
                        preferred_element_type=jnp.float32) + b2_ref[...]
            us[pl.ds(i * BR, BR), :] = u

            @pl.when(i == 0)
            def _():
                su[...] = jnp.zeros_like(su)

            su[0:1, :] += jnp.sum(u, axis=0, keepdims=True)
            su[1:2, :] += jnp.sum(u * u, axis=0, keepdims=True)

        @pl.when(p == 2)
        def _():
            u = us[pl.ds(i * BR, BR), :]
            sv = su[...]
            mean = sv[0:1, :] * (1.0 / N)
            var = sv[1:2, :] * (1.0 / N) - mean * mean
            scale = go_ref[...] * lax.rsqrt(var + EPS)
            shift = beo_ref[...] - mean * scale
            hb = u * scale + shift
            if final:
                h_ref[...] = hb
                bb = bat_ref[0]
                onehot_t = (lax.broadcasted_iota(jnp.int32, (G, BR), 0)
                            == bb).astype(jnp.float32)

                @pl.when(i == 0)
                def _():
                    xp_ref[...] = jnp.zeros_like(xp_ref)

                xp_ref[...] += jnp.dot(onehot_t, hb,
                                       preferred_element_type=jnp.float32)
            else:
                hb = jnp.maximum(hb, 0.0)
                for k in range(4):
                    o_ref[k] = hb[:, k * HQ:(k + 1) * HQ]

    return body


def _const2(pp, ii):
    return (0, 0)


@functools.cache
def _mlp_call(final):
    in_specs = [
        pl.BlockSpec((4, BR, HQ), lambda p, i: (0, jnp.where(p < 2, i, 0), 0)),
        pl.BlockSpec((H, 2 * H), _const2),
        pl.BlockSpec((1, 2 * H), _const2),
        pl.BlockSpec((1, 2 * H), _const2),
        pl.BlockSpec((1, 2 * H), _const2),
        pl.BlockSpec((2 * H, H), _const2),
        pl.BlockSpec((1, H), _const2),
        pl.BlockSpec((1, H), _const2),
        pl.BlockSpec((1, H), _const2),
    ]
    if final:
        in_specs.append(
            pl.BlockSpec((1, 1, BR),
                         lambda p, i: (jnp.where(p == 2, i, 0), 0, 0)))
        out_specs = [
            pl.BlockSpec((BR, H), lambda p, i: (jnp.where(p == 2, i, 0), 0)),
            pl.BlockSpec((G, H), _const2),
        ]
        out_shape = [
            jax.ShapeDtypeStruct((N, H), jnp.float32),
            jax.ShapeDtypeStruct((G, H), jnp.float32),
        ]
    else:
        out_specs = pl.BlockSpec(
            (4, BR, HQ), lambda p, i: (0, jnp.where(p == 2, i, 0), 0))
        out_shape = jax.ShapeDtypeStruct((4, NP, HQ), jnp.float32)
    return pl.pallas_call(
        _mlp_body_maker(final),
        grid=(3, NR),
        in_specs=in_specs,
        out_specs=out_specs,
        out_shape=out_shape,
        scratch_shapes=[
            pltpu.VMEM((N, H), jnp.float32),
            pltpu.VMEM((2, 2 * H), jnp.float32),
            pltpu.VMEM((2, H), jnp.float32),
        ],
    )


def kernel(x, edge_attr, lin_w, lin_b, W1, b1, g1, be1, W2, b2, gO, beO,
           batch, edge_index):
    src = edge_index[0]
    dst = edge_index[1]
    pad = E_PAD - E
    srcp = jnp.pad(src, (0, pad))
    dstp = jnp.pad(dst, (0, pad))
    wp = jnp.pad(edge_attr, (0, pad))
    src3 = srcp.reshape(NB, 128)
    dst3 = dstp.reshape(NB, 128)
    batch3 = batch.reshape(NR, 1, BR)

    hs = _lin(x, lin_w, lin_b.reshape(1, H))
    for i in range(L):
        agg = _spmm(hs, src3, dst3, wp)
        args = (agg, W1[i], b1[i].reshape(1, 2 * H), g1[i].reshape(1, 2 * H),
                be1[i].reshape(1, 2 * H), W2[i], b2[i].reshape(1, H),
                gO[i].reshape(1, H), beO[i].reshape(1, H))
        if i < L - 1:
            hs = _mlp_call(False)(*args)
        else:
            h, xpool = _mlp_call(True)(*args, batch3)
    return h, xpool


# speedup vs baseline: 1.2155x; 1.0205x over previous
import functools

import jax
import jax.numpy as jnp
from jax import lax
from jax.experimental import pallas as pl
from jax.experimental.pallas import tpu as pltpu
from jax.experimental.pallas import tpu_sc as plsc

N = 50000
E = 800000
P = 10
H = 64
HQ = 16
L = 5
G = 128

NS = 16
K = 2
C = K * 128
E_PAD = 819200
NB = E_PAD // 128
NBT = NB // NS
NCH = NBT // K
SCH = 20
NSC = NCH // SCH
NP = 50048
RPT = NP // NS
ZR = 184

BR = 2000
NR = N // BR
EPS = 1e-5

_BCAST_DNUMS = lax.GatherDimensionNumbers(
    offset_dims=(), collapsed_slice_dims=(0,), start_index_map=(0,))


def _lane_bcast(v16, lane):
    idx = jnp.full((16, 1), lane, jnp.int32)
    return lax.gather(v16, idx, _BCAST_DNUMS, (1,),
                      mode=lax.GatherScatterMode.PROMISE_IN_BOUNDS)


def _spmm_body(hs_ref, src3_ref, dst3_ref, w_ref, out_ref,
               rows0, rows1, sstage, dstage, wstage, stage, stage2,
               tab, agg, gsem0, gsem1, ssem0, ssem1):
    c = lax.axis_index("c")
    s = lax.axis_index("s")
    rows = (rows0, rows1)
    gsem = (gsem0, gsem1)
    ssem = (ssem0, ssem1)
    z16 = jnp.zeros((16,), jnp.float32)

    def fire_gather(bi, t):
        for j in range(K):
            pltpu.async_copy(tab.at[sstage.at[t * K + j]],
                             rows[bi].at[pl.ds(j * 128, 128), :], gsem[bi])

    def drain_gather(bi, t):
        for j in range(K):
            pltpu.make_async_copy(
                tab.at[sstage.at[t * K + j]],
                rows[bi].at[pl.ds(j * 128, 128), :], gsem[bi]).wait()

    def fire_scatter(bi, t):
        for j in range(K):
            pltpu.async_copy(rows[bi].at[pl.ds(j * 128, 128), :],
                             agg.at[dstage.at[t * K + j]], ssem[bi], add=True)

    def drain_scatter(bi, t):
        for j in range(K):
            pltpu.make_async_copy(
                rows[bi].at[pl.ds(j * 128, 128), :],
                agg.at[dstage.at[t * K + j]], ssem[bi]).wait()

    def multiply(bi, t):
        rw = rows[bi]
        woff = t * C

        @plsc.parallel_loop(0, C // 16, 1, unroll=2)
        def _(g):
            w16 = wstage[pl.ds(woff + g * 16, 16)]
            for e in range(16):
                wv = _lane_bcast(w16, e)
                r = g * 16 + e
                rw[r, pl.ds(0, 16)] = rw[r, pl.ds(0, 16)] * wv

    for p in range(2):
        tabcp = pltpu.async_copy(hs_ref.at[2 * c + p, pl.ds(s * RPT, RPT), :],
                                 tab.at[pl.ds(s * RPT, RPT), :], ssem0)

        def zloop(r, _):
            stage[r, pl.ds(0, 16)] = z16
            return 0

        lax.fori_loop(0, ZR, zloop, 0)
        zcps = [pltpu.async_copy(stage,
                                 agg.at[pl.ds(s * RPT + q * ZR, ZR), :], gsem0)
                for q in range(RPT // ZR)]
        tabcp.wait()
        for cp in zcps:
            cp.wait()
        plsc.subcore_barrier()

        def superchunk(sc, _):
            base_blk = s * NBT + sc * (SCH * K)
            pltpu.sync_copy(src3_ref.at[pl.ds(base_blk, SCH * K), :], sstage)
            pltpu.sync_copy(dst3_ref.at[pl.ds(base_blk, SCH * K), :], dstage)
            pltpu.sync_copy(w_ref.at[pl.ds(base_blk * 128, SCH * C)], wstage)

            def pair(q, _):
                for b in range(2):
                    t2 = q * 2 + b

                    @pl.when(jnp.logical_and(t2 >= 2, t2 < SCH))
                    def _():
                        drain_scatter(b, t2 - 2)

                    @pl.when(t2 < SCH)
                    def _():
                        fire_gather(b, t2)

                    @pl.when(jnp.logical_and(t2 >= 1, t2 <= SCH))
                    def _():
                        drain_gather(1 - b, t2 - 1)
                        multiply(1 - b, t2 - 1)
                        fire_scatter(1 - b, t2 - 1)
                return 0

            lax.fori_loop(0, SCH // 2 + 1, pair, 0)
            drain_scatter(0, SCH - 2)
            drain_scatter(1, SCH - 1)
            return 0

        lax.fori_loop(0, NSC, superchunk, 0)
        plsc.subcore_barrier()

        stcps = [None, None]
        sts = (stage, stage2)
        for q in range(RPT // ZR):
            b = q % 2
            base = s * RPT + q * ZR
            if stcps[b] is not None:
                stcps[b].wait()
            pltpu.sync_copy(agg.at[pl.ds(base, ZR), :], sts[b])
            stcps[b] = pltpu.async_copy(
                sts[b], out_ref.at[2 * c + p, pl.ds(base, ZR), :], ssem[b])
        for cp in stcps:
            cp.wait()
        plsc.subcore_barrier()


@functools.cache
def _spmm_kernel():
    return pl.kernel(
        _spmm_body,
        out_type=jax.ShapeDtypeStruct((4, NP, HQ), jnp.float32),
        mesh=plsc.VectorSubcoreMesh(core_axis_name="c", subcore_axis_name="s"),
        scratch_types=[
            pltpu.VMEM((C, HQ), jnp.float32),
            pltpu.VMEM((C, HQ), jnp.float32),
            pltpu.VMEM((SCH * K, 128), jnp.int32),
            pltpu.VMEM((SCH * K, 128), jnp.int32),
            pltpu.VMEM((SCH * C,), jnp.float32),
            pltpu.VMEM((ZR, HQ), jnp.float32),
            pltpu.VMEM((ZR, HQ), jnp.float32),
            pltpu.VMEM_SHARED((NP, HQ), jnp.float32),
            pltpu.VMEM_SHARED((NP, HQ), jnp.float32),
            pltpu.SemaphoreType.DMA,
            pltpu.SemaphoreType.DMA,
            pltpu.SemaphoreType.DMA,
            pltpu.SemaphoreType.DMA,
        ],
        compiler_params=pltpu.CompilerParams(use_tc_tiling_on_sc=False),
    )


def _spmm(hs, src3, dst3, wp):
    return _spmm_kernel()(hs, src3, dst3, wp)


def _lin_body(x_ref, w_ref, b_ref, o_ref):
    h = jnp.dot(x_ref[...], w_ref[...],
                preferred_element_type=jnp.float32) + b_ref[...]
    for k in range(4):
        o_ref[k] = h[:, k * HQ:(k + 1) * HQ]


def _lin(x, w, b):
    return pl.pallas_call(
        _lin_body,
        grid=(NR,),
        in_specs=[
            pl.BlockSpec((BR, 2 * P), lambda i: (i, 0)),
            pl.BlockSpec((2 * P, H), lambda i: (0, 0)),
            pl.BlockSpec((1, H), lambda i: (0, 0)),
        ],
        out_specs=pl.BlockSpec((4, BR, HQ), lambda i: (0, i, 0)),
        out_shape=jax.ShapeDtypeStruct((4, NP, HQ), jnp.float32),
    )(x, w, b)


def _mlp_body_maker(final):
    def body(*refs):
        if final:
            (agg_ref, w1_ref, b1_ref, g1_ref, be1_ref, w2_ref, b2_ref,
             go_ref, beo_ref, bat_ref, h_ref, xp_ref, us, st, su) = refs
        else:
            (agg_ref, w1_ref, b1_ref, g1_ref, be1_ref, w2_ref, b2_ref,
             go_ref, beo_ref, o_ref, us, st, su) = refs
        p = pl.program_id(0)
        i = pl.program_id(1)

        def compute_t():
            a = jnp.concatenate([agg_ref[k] for k in range(4)], axis=1)
            return jnp.dot(a, w1_ref[...],
                           preferred_element_type=jnp.float32) + b1_ref[...]

        @pl.when(p == 0)
        def _():
            t = compute_t()

            @pl.when(i == 0)
            def _():
                st[...] = jnp.zeros_like(st)

            st[0:1, :] += jnp.sum(t, axis=0, keepdims=True)
            st[1:2, :] += jnp.sum(t * t, axis=0, keepdims=True)

        @pl.when(p == 1)
        def _():
            t = compute_t()
            sv = st[...]
            mean = sv[0:1, :] * (1.0 / N)
            var = sv[1:2, :] * (1.0 / N) - mean * mean
            scale = g1_ref[...] * lax.rsqrt(var + EPS)
            shift = be1_ref[...] - mean * scale
            tn = jnp.maximum(t * scale + shift, 0.0)
            u = jnp.dot(tn, w2_ref[...],
                        preferred_element_type=jnp.float32) + b2_ref[...]
            us[pl.ds(i * BR, BR), :] = u

            @pl.when(i == 0)
            def _():
                su[...] = jnp.zeros_like(su)

            su[0:1, :] += jnp.sum(u, axis=0, keepdims=True)
            su[1:2, :] += jnp.sum(u * u, axis=0, keepdims=True)

        @pl.when(p == 2)
        def _():
            u = us[pl.ds(i * BR, BR), :]
            sv = su[...]
            mean = sv[0:1, :] * (1.0 / N)
            var = sv[1:2, :] * (1.0 / N) - mean * mean
            scale = go_ref[...] * lax.rsqrt(var + EPS)
            shift = beo_ref[...] - mean * scale
            hb = u * scale + shift
            if final:
                h_ref[...] = hb
                bb = bat_ref[0]
                onehot_t = (lax.broadcasted_iota(jnp.int32, (G, BR), 0)
                            == bb).astype(jnp.float32)

                @pl.when(i == 0)
                def _():
                    xp_ref[...] = jnp.zeros_like(xp_ref)

                xp_ref[...] += jnp.dot(onehot_t, hb,
                                       preferred_element_type=jnp.float32)
            else:
                hb = jnp.maximum(hb, 0.0)
                for k in range(4):
                    o_ref[k] = hb[:, k * HQ:(k + 1) * HQ]

    return body


def _const2(pp, ii):
    return (0, 0)


@functools.cache
def _mlp_call(final):
    in_specs = [
        pl.BlockSpec((4, BR, HQ), lambda p, i: (0, jnp.where(p < 2, i, 0), 0)),
        pl.BlockSpec((H, 2 * H), _const2),
        pl.BlockSpec((1, 2 * H), _const2),
        pl.BlockSpec((1, 2 * H), _const2),
        pl.BlockSpec((1, 2 * H), _const2),
        pl.BlockSpec((2 * H, H), _const2),
        pl.BlockSpec((1, H), _const2),
        pl.BlockSpec((1, H), _const2),
        pl.BlockSpec((1, H), _const2),
    ]
    if final:
        in_specs.append(
            pl.BlockSpec((1, 1, BR),
                         lambda p, i: (jnp.where(p == 2, i, 0), 0, 0)))
        out_specs = [
            pl.BlockSpec((BR, H), lambda p, i: (jnp.where(p == 2, i, 0), 0)),
            pl.BlockSpec((G, H), _const2),
        ]
        out_shape = [
            jax.ShapeDtypeStruct((N, H), jnp.float32),
            jax.ShapeDtypeStruct((G, H), jnp.float32),
        ]
    else:
        out_specs = pl.BlockSpec(
            (4, BR, HQ), lambda p, i: (0, jnp.where(p == 2, i, 0), 0))
        out_shape = jax.ShapeDtypeStruct((4, NP, HQ), jnp.float32)
    return pl.pallas_call(
        _mlp_body_maker(final),
        grid=(3, NR),
        in_specs=in_specs,
        out_specs=out_specs,
        out_shape=out_shape,
        scratch_shapes=[
            pltpu.VMEM((N, H), jnp.float32),
            pltpu.VMEM((2, 2 * H), jnp.float32),
            pltpu.VMEM((2, H), jnp.float32),
        ],
    )


def kernel(x, edge_attr, lin_w, lin_b, W1, b1, g1, be1, W2, b2, gO, beO,
           batch, edge_index):
    src = edge_index[0]
    dst = edge_index[1]
    pad = E_PAD - E
    srcp = jnp.pad(src, (0, pad))
    dstp = jnp.pad(dst, (0, pad))
    wp = jnp.pad(edge_attr, (0, pad))
    src3 = srcp.reshape(NB, 128)
    dst3 = dstp.reshape(NB, 128)
    batch3 = batch.reshape(NR, 1, BR)

    hs = _lin(x, lin_w, lin_b.reshape(1, H))
    for i in range(L):
        agg = _spmm(hs, src3, dst3, wp)
        args = (agg, W1[i], b1[i].reshape(1, 2 * H), g1[i].reshape(1, 2 * H),
                be1[i].reshape(1, 2 * H), W2[i], b2[i].reshape(1, H),
                gO[i].reshape(1, H), beO[i].reshape(1, H))
        if i < L - 1:
            hs = _mlp_call(False)(*args)
        else:
            h, xpool = _mlp_call(True)(*args, batch3)
    return h, xpool
